# Initial kernel scaffold; baseline (speedup 1.0000x reference)
import functools

import jax
import jax.numpy as jnp
from jax import lax
from jax.experimental import pallas as pl
from jax.experimental.pallas import tpu as pltpu
from jax.experimental.pallas import tpu_sc as plsc

N_NODES = 10000
N_PAD = 10240
D = 256
DH = 128
E_PAD = 163840
EB = 128
NC, NS = 2, 16
ACC_ROWS = 10368
DEG_SIZE = 10496
RB = 1024
GR = N_PAD // RB


def _elu(v):
    return jnp.where(v > 0, v, jnp.exp(jnp.minimum(v, 0.0)) - 1.0)


def _deg_body(dst_hbm, deg_out, idx_v, ones_v, zb_v, deg_sp):
    c = lax.axis_index("c")
    t = lax.axis_index("s")

    def fill_z(i, _):
        zb_v[pl.ds(i * 16, 16)] = jnp.zeros((16,), jnp.float32)
        return 0

    lax.fori_loop(0, 656 // 16, fill_z, 0)

    def fill_o(i, _):
        ones_v[pl.ds(i * 16, 16)] = jnp.ones((16,), jnp.float32)
        return 0

    lax.fori_loop(0, EB // 16, fill_o, 0)

    pltpu.sync_copy(zb_v, deg_sp.at[pl.ds(t * 656, 656)])
    plsc.subcore_barrier()

    base = (c * NS + t) * (E_PAD // (NC * NS))

    def body(i, _):
        pltpu.sync_copy(dst_hbm.at[pl.ds(base + i * EB, EB)], idx_v)
        pltpu.sync_copy(ones_v, deg_sp.at[idx_v], add=True)
        return 0

    lax.fori_loop(0, (E_PAD // (NC * NS)) // EB, body, 0)
    plsc.subcore_barrier()
    pltpu.sync_copy(deg_sp.at[pl.ds(t * 656, 656)],
                    deg_out.at[pl.ds(c * DEG_SIZE + t * 656, 656)])


_deg_kernel = pl.kernel(
    _deg_body,
    out_type=jax.ShapeDtypeStruct((NC * DEG_SIZE,), jnp.float32),
    mesh=plsc.VectorSubcoreMesh(core_axis_name="c", subcore_axis_name="s"),
    scratch_types=[
        pltpu.VMEM((EB,), jnp.int32),
        pltpu.VMEM((EB,), jnp.float32),
        pltpu.VMEM((656,), jnp.float32),
        pltpu.VMEM_SHARED((DEG_SIZE,), jnp.float32),
    ],
)


def _agg_body(s_hbm, srcx_hbm, dst_hbm, acc_out, sidx, didx, rows, acc_sp):
    c = lax.axis_index("c")
    t = lax.axis_index("s")

    def fill_z(i, _):
        for j in range(DH // 16):
            rows[i, pl.ds(j * 16, 16)] = jnp.zeros((16,), jnp.float32)
        return 0

    lax.fori_loop(0, EB, fill_z, 0)

    for k in range(5):
        pltpu.sync_copy(rows, acc_sp.at[pl.ds(t * 648 + k * EB, EB)])
    pltpu.sync_copy(rows.at[pl.ds(0, 8)], acc_sp.at[pl.ds(t * 648 + 640, 8)])
    plsc.subcore_barrier()

    epw = E_PAD // NS
    base = t * epw

    def body(i, _):
        e0 = base + i * EB
        pltpu.sync_copy(srcx_hbm.at[pl.ds(c * E_PAD + e0, EB)], sidx)
        pltpu.sync_copy(dst_hbm.at[pl.ds(e0, EB)], didx)
        pltpu.sync_copy(s_hbm.at[sidx], rows)
        pltpu.sync_copy(rows, acc_sp.at[didx], add=True)
        return 0

    lax.fori_loop(0, epw // EB, body, 0)
    plsc.subcore_barrier()
    pltpu.sync_copy(acc_sp.at[pl.ds(t * 640, 640)],
                    acc_out.at[pl.ds(c * N_PAD + t * 640, 640)])


_agg_kernel = pl.kernel(
    _agg_body,
    out_type=jax.ShapeDtypeStruct((NC * N_PAD, DH), jnp.float32),
    mesh=plsc.VectorSubcoreMesh(core_axis_name="c", subcore_axis_name="s"),
    scratch_types=[
        pltpu.VMEM((EB,), jnp.int32),
        pltpu.VMEM((EB,), jnp.int32),
        pltpu.VMEM((EB, DH), jnp.float32),
        pltpu.VMEM_SHARED((ACC_ROWS, DH), jnp.float32),
    ],
)


def _k1_body(x_ref, w_ref, degp_ref, s_ref, dinv_ref):
    deg = degp_ref[0, :] + degp_ref[1, :] + 1.0
    dinv = lax.rsqrt(deg)
    z = jnp.dot(x_ref[...], w_ref[...], preferred_element_type=jnp.float32)
    s_ref[...] = z * dinv[:, None]
    dinv_ref[...] = dinv[None, :]


def _k1(x_p, W1, degp):
    return pl.pallas_call(
        _k1_body,
        grid=(GR, 2),
        in_specs=[
            pl.BlockSpec((RB, D), lambda i, j: (i, 0)),
            pl.BlockSpec((D, DH), lambda i, j: (0, j)),
            pl.BlockSpec((2, RB), lambda i, j: (0, i)),
        ],
        out_specs=[
            pl.BlockSpec((RB, DH), lambda i, j: (j * GR + i, 0)),
            pl.BlockSpec((1, RB), lambda i, j: (0, i)),
        ],
        out_shape=[
            jax.ShapeDtypeStruct((NC * N_PAD, DH), jnp.float32),
            jax.ShapeDtypeStruct((1, N_PAD), jnp.float32),
        ],
    )(x_p, W1, degp)


def _k2_body(accA, accB, sA, sB, dinv_ref, b_ref, w_ref, out_ref):
    di = dinv_ref[0, :]
    hA = _elu(di[:, None] * (accA[...] + sA[...]) + b_ref[0, 0:DH][None, :])
    hB = _elu(di[:, None] * (accB[...] + sB[...]) + b_ref[0, DH:D][None, :])
    h = jnp.concatenate([hA, hB], axis=1)
    z = jnp.dot(h, w_ref[...], preferred_element_type=jnp.float32)
    out_ref[...] = z * di[:, None]


def _k2(acc1, s1, dinv, b1, W2):
    blk = pl.BlockSpec((RB, DH), lambda i, j: (i, 0))
    blkB = pl.BlockSpec((RB, DH), lambda i, j: (GR + i, 0))
    return pl.pallas_call(
        _k2_body,
        grid=(GR, 2),
        in_specs=[
            blk, blkB, blk, blkB,
            pl.BlockSpec((1, RB), lambda i, j: (0, i)),
            pl.BlockSpec((1, D), lambda i, j: (0, 0)),
            pl.BlockSpec((D, DH), lambda i, j: (0, j)),
        ],
        out_specs=pl.BlockSpec((RB, DH), lambda i, j: (j * GR + i, 0)),
        out_shape=jax.ShapeDtypeStruct((NC * N_PAD, DH), jnp.float32),
    )(acc1, acc1, s1, s1, dinv, b1, W2)


def _k3_body(accA, accB, sA, sB, dinv_ref, b_ref, out_ref):
    di = dinv_ref[0, :]
    hA = _elu(di[:, None] * (accA[...] + sA[...]) + b_ref[0, 0:DH][None, :])
    hB = _elu(di[:, None] * (accB[...] + sB[...]) + b_ref[0, DH:D][None, :])
    out_ref[...] = jnp.concatenate([hA, hB], axis=1)


def _k3(acc2, s2, dinv, b2):
    blk = pl.BlockSpec((RB, DH), lambda i: (i, 0))
    blkB = pl.BlockSpec((RB, DH), lambda i: (GR + i, 0))
    return pl.pallas_call(
        _k3_body,
        grid=(GR,),
        in_specs=[
            blk, blkB, blk, blkB,
            pl.BlockSpec((1, RB), lambda i: (0, i)),
            pl.BlockSpec((1, D), lambda i: (0, 0)),
        ],
        out_specs=pl.BlockSpec((RB, D), lambda i: (i, 0)),
        out_shape=jax.ShapeDtypeStruct((N_PAD, D), jnp.float32),
    )(acc2, acc2, s2, s2, dinv, b2)


@jax.jit
def _run(x, src, dst, W1, b1, W2, b2):
    n_e = src.shape[0]
    pe = E_PAD - n_e
    src_p = jnp.concatenate([src, jnp.zeros((pe,), jnp.int32)])
    dst_p = jnp.concatenate([dst, jnp.full((pe,), N_PAD, jnp.int32)])
    srcx = jnp.concatenate([src_p, src_p + N_PAD])
    x_p = jnp.pad(x, ((0, N_PAD - x.shape[0]), (0, 0)))

    degp = _deg_kernel(dst_p).reshape(NC, DEG_SIZE)[:, :N_PAD]
    s1, dinv = _k1(x_p, W1, degp)
    acc1 = _agg_kernel(s1, srcx, dst_p)
    s2 = _k2(acc1, s1, dinv, b1[None, :], W2)
    acc2 = _agg_kernel(s2, srcx, dst_p)
    out = _k3(acc2, s2, dinv, b2[None, :])
    return out[:N_NODES]


def kernel(x, edge_index, W1, b1, W2, b2):
    src = edge_index[0].astype(jnp.int32)
    dst = edge_index[1].astype(jnp.int32)
    return _run(x, src, dst, W1, b1, W2, b2)


# speedup vs baseline: 6.0734x; 6.0734x over previous
import functools

import jax
import jax.numpy as jnp
from jax import lax
from jax.experimental import pallas as pl
from jax.experimental.pallas import tpu as pltpu
from jax.experimental.pallas import tpu_sc as plsc

N_NODES = 10000
N_PAD = 10240
D = 256
DH = 128
E_PAD = 163840
EB = 128
NC, NS = 2, 16
ACC_ROWS = 10368
DEG_SIZE = 10496
RB = 1024
GR = N_PAD // RB


def _elu(v):
    return jnp.where(v > 0, v, jnp.exp(jnp.minimum(v, 0.0)) - 1.0)


def _deg_body(dst_hbm, deg_out, idx_v, ones_v, zb_v, deg_sp):
    c = lax.axis_index("c")
    t = lax.axis_index("s")

    def fill_z(i, _):
        zb_v[pl.ds(i * 16, 16)] = jnp.zeros((16,), jnp.float32)
        return 0

    lax.fori_loop(0, 656 // 16, fill_z, 0)

    def fill_o(i, _):
        ones_v[pl.ds(i * 16, 16)] = jnp.ones((16,), jnp.float32)
        return 0

    lax.fori_loop(0, EB // 16, fill_o, 0)

    pltpu.sync_copy(zb_v, deg_sp.at[pl.ds(t * 656, 656)])
    plsc.subcore_barrier()

    base = (c * NS + t) * (E_PAD // (NC * NS))

    def body(i, _):
        pltpu.sync_copy(dst_hbm.at[pl.ds(base + i * EB, EB)], idx_v)
        pltpu.sync_copy(ones_v, deg_sp.at[idx_v], add=True)
        return 0

    lax.fori_loop(0, (E_PAD // (NC * NS)) // EB, body, 0)
    plsc.subcore_barrier()
    pltpu.sync_copy(deg_sp.at[pl.ds(t * 656, 656)], zb_v)
    pltpu.sync_copy(zb_v, deg_out.at[pl.ds(c * DEG_SIZE + t * 656, 656)])


_deg_kernel = pl.kernel(
    _deg_body,
    out_type=jax.ShapeDtypeStruct((NC * DEG_SIZE,), jnp.float32),
    mesh=plsc.VectorSubcoreMesh(core_axis_name="c", subcore_axis_name="s"),
    scratch_types=[
        pltpu.VMEM((EB,), jnp.int32),
        pltpu.VMEM((EB,), jnp.float32),
        pltpu.VMEM((656,), jnp.float32),
        pltpu.VMEM_SHARED((DEG_SIZE,), jnp.float32),
    ],
)


def _agg_body(s_hbm, srcx_hbm, dst_hbm, acc_out, sidx, didx, rows, acc_sp):
    c = lax.axis_index("c")
    t = lax.axis_index("s")

    def fill_z(i, _):
        for j in range(DH // 16):
            rows[i, pl.ds(j * 16, 16)] = jnp.zeros((16,), jnp.float32)
        return 0

    lax.fori_loop(0, EB, fill_z, 0)

    for k in range(5):
        pltpu.sync_copy(rows, acc_sp.at[pl.ds(t * 648 + k * EB, EB)])
    pltpu.sync_copy(rows.at[pl.ds(0, 8)], acc_sp.at[pl.ds(t * 648 + 640, 8)])
    plsc.subcore_barrier()

    epw = E_PAD // NS
    base = t * epw

    def body(i, _):
        e0 = base + i * EB
        pltpu.sync_copy(srcx_hbm.at[pl.ds(c * E_PAD + e0, EB)], sidx)
        pltpu.sync_copy(dst_hbm.at[pl.ds(e0, EB)], didx)
        pltpu.sync_copy(s_hbm.at[sidx], rows)
        pltpu.sync_copy(rows, acc_sp.at[didx], add=True)
        return 0

    lax.fori_loop(0, epw // EB, body, 0)
    plsc.subcore_barrier()
    pltpu.sync_copy(acc_sp.at[pl.ds(t * 640, 640)],
                    acc_out.at[pl.ds(c * N_PAD + t * 640, 640)])


_agg_kernel = pl.kernel(
    _agg_body,
    out_type=jax.ShapeDtypeStruct((NC * N_PAD, DH), jnp.float32),
    mesh=plsc.VectorSubcoreMesh(core_axis_name="c", subcore_axis_name="s"),
    scratch_types=[
        pltpu.VMEM((EB,), jnp.int32),
        pltpu.VMEM((EB,), jnp.int32),
        pltpu.VMEM((EB, DH), jnp.float32),
        pltpu.VMEM_SHARED((ACC_ROWS, DH), jnp.float32),
    ],
)


def _k1_body(x_ref, w_ref, degp_ref, s_ref, dinv_ref):
    deg = degp_ref[0, :] + degp_ref[1, :] + 1.0
    dinv = lax.rsqrt(deg)
    z = jnp.dot(x_ref[...], w_ref[...], preferred_element_type=jnp.float32)
    s_ref[...] = z * dinv[:, None]
    dinv_ref[...] = dinv[None, :]


def _k1(x_p, W1, degp):
    return pl.pallas_call(
        _k1_body,
        grid=(GR, 2),
        in_specs=[
            pl.BlockSpec((RB, D), lambda i, j: (i, 0)),
            pl.BlockSpec((D, DH), lambda i, j: (0, j)),
            pl.BlockSpec((2, RB), lambda i, j: (0, i)),
        ],
        out_specs=[
            pl.BlockSpec((RB, DH), lambda i, j: (j * GR + i, 0)),
            pl.BlockSpec((1, RB), lambda i, j: (0, i)),
        ],
        out_shape=[
            jax.ShapeDtypeStruct((NC * N_PAD, DH), jnp.float32),
            jax.ShapeDtypeStruct((1, N_PAD), jnp.float32),
        ],
    )(x_p, W1, degp)


def _k2_body(accA, accB, sA, sB, dinv_ref, b_ref, w_ref, out_ref):
    di = dinv_ref[0, :]
    hA = _elu(di[:, None] * (accA[...] + sA[...]) + b_ref[0, 0:DH][None, :])
    hB = _elu(di[:, None] * (accB[...] + sB[...]) + b_ref[0, DH:D][None, :])
    h = jnp.concatenate([hA, hB], axis=1)
    z = jnp.dot(h, w_ref[...], preferred_element_type=jnp.float32)
    out_ref[...] = z * di[:, None]


def _k2(acc1, s1, dinv, b1, W2):
    blk = pl.BlockSpec((RB, DH), lambda i, j: (i, 0))
    blkB = pl.BlockSpec((RB, DH), lambda i, j: (GR + i, 0))
    return pl.pallas_call(
        _k2_body,
        grid=(GR, 2),
        in_specs=[
            blk, blkB, blk, blkB,
            pl.BlockSpec((1, RB), lambda i, j: (0, i)),
            pl.BlockSpec((1, D), lambda i, j: (0, 0)),
            pl.BlockSpec((D, DH), lambda i, j: (0, j)),
        ],
        out_specs=pl.BlockSpec((RB, DH), lambda i, j: (j * GR + i, 0)),
        out_shape=jax.ShapeDtypeStruct((NC * N_PAD, DH), jnp.float32),
    )(acc1, acc1, s1, s1, dinv, b1, W2)


def _k3_body(accA, accB, sA, sB, dinv_ref, b_ref, out_ref):
    di = dinv_ref[0, :]
    hA = _elu(di[:, None] * (accA[...] + sA[...]) + b_ref[0, 0:DH][None, :])
    hB = _elu(di[:, None] * (accB[...] + sB[...]) + b_ref[0, DH:D][None, :])
    out_ref[...] = jnp.concatenate([hA, hB], axis=1)


def _k3(acc2, s2, dinv, b2):
    blk = pl.BlockSpec((RB, DH), lambda i: (i, 0))
    blkB = pl.BlockSpec((RB, DH), lambda i: (GR + i, 0))
    return pl.pallas_call(
        _k3_body,
        grid=(GR,),
        in_specs=[
            blk, blkB, blk, blkB,
            pl.BlockSpec((1, RB), lambda i: (0, i)),
            pl.BlockSpec((1, D), lambda i: (0, 0)),
        ],
        out_specs=pl.BlockSpec((RB, D), lambda i: (i, 0)),
        out_shape=jax.ShapeDtypeStruct((N_PAD, D), jnp.float32),
    )(acc2, acc2, s2, s2, dinv, b2)


@jax.jit
def _run(x, src, dst, W1, b1, W2, b2):
    n_e = src.shape[0]
    pe = E_PAD - n_e
    src_p = jnp.concatenate([src, jnp.zeros((pe,), jnp.int32)])
    dst_p = jnp.concatenate([dst, jnp.full((pe,), N_PAD, jnp.int32)])
    srcx = jnp.concatenate([src_p, src_p + N_PAD])
    x_p = jnp.pad(x, ((0, N_PAD - x.shape[0]), (0, 0)))

    degp = _deg_kernel(dst_p).reshape(NC, DEG_SIZE)[:, :N_PAD]
    s1, dinv = _k1(x_p, W1, degp)
    acc1 = _agg_kernel(s1, srcx, dst_p)
    s2 = _k2(acc1, s1, dinv, b1[None, :], W2)
    acc2 = _agg_kernel(s2, srcx, dst_p)
    out = _k3(acc2, s2, dinv, b2[None, :])
    return out[:N_NODES]


def kernel(x, edge_index, W1, b1, W2, b2):
    src = edge_index[0].astype(jnp.int32)
    dst = edge_index[1].astype(jnp.int32)
    return _run(x, src, dst, W1, b1, W2, b2)


# speedup vs baseline: 7.2984x; 1.2017x over previous
import functools

import jax
import jax.numpy as jnp
from jax import lax
from jax.experimental import pallas as pl
from jax.experimental.pallas import tpu as pltpu
from jax.experimental.pallas import tpu_sc as plsc

N_NODES = 10000
N_PAD = 10240
D = 256
DH = 128
E_PAD = 163840
EB = 80
NC, NS = 2, 16
ACC_ROWS = 10368
DEG_SIZE = 10496
RB = 1024
GR = N_PAD // RB


def _elu(v):
    return jnp.where(v > 0, v, jnp.exp(jnp.minimum(v, 0.0)) - 1.0)


def _deg_body(dst_hbm, deg_out, idx_v, ones_v, zb_v, deg_sp):
    c = lax.axis_index("c")
    t = lax.axis_index("s")

    def fill_z(i, _):
        zb_v[pl.ds(i * 16, 16)] = jnp.zeros((16,), jnp.float32)
        return 0

    lax.fori_loop(0, 656 // 16, fill_z, 0)

    def fill_o(i, _):
        ones_v[pl.ds(i * 16, 16)] = jnp.ones((16,), jnp.float32)
        return 0

    lax.fori_loop(0, EB // 16, fill_o, 0)

    pltpu.sync_copy(zb_v, deg_sp.at[pl.ds(t * 656, 656)])
    plsc.subcore_barrier()

    base = (c * NS + t) * (E_PAD // (NC * NS))

    def body(i, _):
        pltpu.sync_copy(dst_hbm.at[pl.ds(base + i * EB, EB)], idx_v)
        pltpu.sync_copy(ones_v, deg_sp.at[idx_v], add=True)
        return 0

    lax.fori_loop(0, (E_PAD // (NC * NS)) // EB, body, 0)
    plsc.subcore_barrier()
    pltpu.sync_copy(deg_sp.at[pl.ds(t * 656, 656)], zb_v)
    pltpu.sync_copy(zb_v, deg_out.at[pl.ds(c * DEG_SIZE + t * 656, 656)])


_deg_kernel = pl.kernel(
    _deg_body,
    out_type=jax.ShapeDtypeStruct((NC * DEG_SIZE,), jnp.float32),
    mesh=plsc.VectorSubcoreMesh(core_axis_name="c", subcore_axis_name="s"),
    scratch_types=[
        pltpu.VMEM((EB,), jnp.int32),
        pltpu.VMEM((EB,), jnp.float32),
        pltpu.VMEM((656,), jnp.float32),
        pltpu.VMEM_SHARED((DEG_SIZE,), jnp.float32),
    ],
)


NB = (E_PAD // NS) // EB


def _agg_body(s_hbm, srcx_hbm, dst_hbm, acc_out, sidx, dflat, didx0, didx1,
              rows0, rows1, acc_sp, gsem, ssem):
    c = lax.axis_index("c")
    t = lax.axis_index("s")
    epw = E_PAD // NS

    pltpu.sync_copy(srcx_hbm.at[pl.ds((c * NS + t) * epw, epw)], sidx)
    pltpu.sync_copy(dst_hbm.at[pl.ds(t * epw, epw)], dflat)

    def fill_z(i, _):
        for j in range(DH // 16):
            rows0[i, pl.ds(j * 16, 16)] = jnp.zeros((16,), jnp.float32)
        return 0

    lax.fori_loop(0, EB, fill_z, 0)

    for k in range(8):
        pltpu.sync_copy(rows0, acc_sp.at[pl.ds(t * 648 + k * EB, EB)])
    pltpu.sync_copy(rows0.at[pl.ds(0, 8)], acc_sp.at[pl.ds(t * 648 + 640, 8)])
    plsc.subcore_barrier()

    def start_g(i, buf):
        pltpu.async_copy(s_hbm.at[sidx.at[pl.ds(i * EB, EB)]], buf, gsem)

    def wait_g(buf):
        pltpu.make_async_copy(s_hbm.at[sidx.at[pl.ds(0, EB)]], buf, gsem).wait()

    def repack_d(i, dbuf):
        for j in range(EB // 16):
            dbuf[pl.ds(j * 16, 16)] = dflat[pl.ds(i * EB + j * 16, 16)]

    def start_s(dbuf, buf):
        pltpu.async_copy(buf, acc_sp.at[dbuf], ssem, add=True)

    def wait_s(dbuf, buf):
        pltpu.make_async_copy(buf, acc_sp.at[dbuf], ssem).wait()

    start_g(0, rows0)

    def body(k, _):
        wait_g(rows0)

        @pl.when(k > 0)
        def _():
            wait_s(didx1, rows1)

        start_g(2 * k + 1, rows1)
        repack_d(2 * k, didx0)
        start_s(didx0, rows0)
        wait_g(rows1)
        wait_s(didx0, rows0)

        @pl.when(k < NB // 2 - 1)
        def _():
            start_g(2 * k + 2, rows0)

        repack_d(2 * k + 1, didx1)
        start_s(didx1, rows1)
        return 0

    lax.fori_loop(0, NB // 2, body, 0)
    wait_s(didx1, rows1)
    plsc.subcore_barrier()
    pltpu.sync_copy(acc_sp.at[pl.ds(t * 640, 640)],
                    acc_out.at[pl.ds(c * N_PAD + t * 640, 640)])


_agg_kernel = pl.kernel(
    _agg_body,
    out_type=jax.ShapeDtypeStruct((NC * N_PAD, DH), jnp.float32),
    mesh=plsc.VectorSubcoreMesh(core_axis_name="c", subcore_axis_name="s"),
    scratch_types=[
        pltpu.VMEM((E_PAD // NS,), jnp.int32),
        pltpu.VMEM((E_PAD // NS,), jnp.int32),
        pltpu.VMEM((EB,), jnp.int32),
        pltpu.VMEM((EB,), jnp.int32),
        pltpu.VMEM((EB, DH), jnp.float32),
        pltpu.VMEM((EB, DH), jnp.float32),
        pltpu.VMEM_SHARED((ACC_ROWS, DH), jnp.float32),
        pltpu.SemaphoreType.DMA,
        pltpu.SemaphoreType.DMA,
    ],
)


def _k1_body(x_ref, w_ref, degp_ref, s_ref, dinv_ref):
    deg = degp_ref[0, :] + degp_ref[1, :] + 1.0
    dinv = lax.rsqrt(deg)
    z = jnp.dot(x_ref[...], w_ref[...], preferred_element_type=jnp.float32)
    s_ref[...] = z * dinv[:, None]
    dinv_ref[...] = dinv[None, :]


def _k1(x_p, W1, degp):
    return pl.pallas_call(
        _k1_body,
        grid=(GR, 2),
        in_specs=[
            pl.BlockSpec((RB, D), lambda i, j: (i, 0)),
            pl.BlockSpec((D, DH), lambda i, j: (0, j)),
            pl.BlockSpec((2, RB), lambda i, j: (0, i)),
        ],
        out_specs=[
            pl.BlockSpec((RB, DH), lambda i, j: (j * GR + i, 0)),
            pl.BlockSpec((1, RB), lambda i, j: (0, i)),
        ],
        out_shape=[
            jax.ShapeDtypeStruct((NC * N_PAD, DH), jnp.float32),
            jax.ShapeDtypeStruct((1, N_PAD), jnp.float32),
        ],
    )(x_p, W1, degp)


def _k2_body(accA, accB, sA, sB, dinv_ref, b_ref, w_ref, out_ref):
    di = dinv_ref[0, :]
    hA = _elu(di[:, None] * (accA[...] + sA[...]) + b_ref[0, 0:DH][None, :])
    hB = _elu(di[:, None] * (accB[...] + sB[...]) + b_ref[0, DH:D][None, :])
    h = jnp.concatenate([hA, hB], axis=1)
    z = jnp.dot(h, w_ref[...], preferred_element_type=jnp.float32)
    out_ref[...] = z * di[:, None]


def _k2(acc1, s1, dinv, b1, W2):
    blk = pl.BlockSpec((RB, DH), lambda i, j: (i, 0))
    blkB = pl.BlockSpec((RB, DH), lambda i, j: (GR + i, 0))
    return pl.pallas_call(
        _k2_body,
        grid=(GR, 2),
        in_specs=[
            blk, blkB, blk, blkB,
            pl.BlockSpec((1, RB), lambda i, j: (0, i)),
            pl.BlockSpec((1, D), lambda i, j: (0, 0)),
            pl.BlockSpec((D, DH), lambda i, j: (0, j)),
        ],
        out_specs=pl.BlockSpec((RB, DH), lambda i, j: (j * GR + i, 0)),
        out_shape=jax.ShapeDtypeStruct((NC * N_PAD, DH), jnp.float32),
    )(acc1, acc1, s1, s1, dinv, b1, W2)


def _k3_body(accA, accB, sA, sB, dinv_ref, b_ref, out_ref):
    di = dinv_ref[0, :]
    hA = _elu(di[:, None] * (accA[...] + sA[...]) + b_ref[0, 0:DH][None, :])
    hB = _elu(di[:, None] * (accB[...] + sB[...]) + b_ref[0, DH:D][None, :])
    out_ref[...] = jnp.concatenate([hA, hB], axis=1)


def _k3(acc2, s2, dinv, b2):
    blk = pl.BlockSpec((RB, DH), lambda i: (i, 0))
    blkB = pl.BlockSpec((RB, DH), lambda i: (GR + i, 0))
    return pl.pallas_call(
        _k3_body,
        grid=(GR,),
        in_specs=[
            blk, blkB, blk, blkB,
            pl.BlockSpec((1, RB), lambda i: (0, i)),
            pl.BlockSpec((1, D), lambda i: (0, 0)),
        ],
        out_specs=pl.BlockSpec((RB, D), lambda i: (i, 0)),
        out_shape=jax.ShapeDtypeStruct((N_PAD, D), jnp.float32),
    )(acc2, acc2, s2, s2, dinv, b2)


@jax.jit
def _run(x, src, dst, W1, b1, W2, b2):
    n_e = src.shape[0]
    pe = E_PAD - n_e
    src_p = jnp.concatenate([src, jnp.zeros((pe,), jnp.int32)])
    dst_p = jnp.concatenate([dst, jnp.full((pe,), N_PAD, jnp.int32)])
    srcx = jnp.concatenate([src_p, src_p + N_PAD])
    x_p = jnp.pad(x, ((0, N_PAD - x.shape[0]), (0, 0)))

    degp = _deg_kernel(dst_p).reshape(NC, DEG_SIZE)[:, :N_PAD]
    s1, dinv = _k1(x_p, W1, degp)
    acc1 = _agg_kernel(s1, srcx, dst_p)
    s2 = _k2(acc1, s1, dinv, b1[None, :], W2)
    acc2 = _agg_kernel(s2, srcx, dst_p)
    out = _k3(acc2, s2, dinv, b2[None, :])
    return out[:N_NODES]


def kernel(x, edge_index, W1, b1, W2, b2):
    src = edge_index[0].astype(jnp.int32)
    dst = edge_index[1].astype(jnp.int32)
    return _run(x, src, dst, W1, b1, W2, b2)


# speedup vs baseline: 8.2566x; 1.1313x over previous
import functools

import jax
import jax.numpy as jnp
from jax import lax
from jax.experimental import pallas as pl
from jax.experimental.pallas import tpu as pltpu
from jax.experimental.pallas import tpu_sc as plsc

N_NODES = 10000
N_PAD = 10240
D = 256
DH = 128
E_PAD = 163840
EB = 80
NC, NS = 2, 16
ACC_ROWS = 10368
DEG_SIZE = 10496
RB = 1024
GR = N_PAD // RB


def _elu(v):
    return jnp.where(v > 0, v, jnp.exp(jnp.minimum(v, 0.0)) - 1.0)


def _deg_body(dst_hbm, deg_out, idx_v, ones_v, zb_v, deg_sp):
    c = lax.axis_index("c")
    t = lax.axis_index("s")

    def fill_z(i, _):
        zb_v[pl.ds(i * 16, 16)] = jnp.zeros((16,), jnp.float32)
        return 0

    lax.fori_loop(0, 656 // 16, fill_z, 0)

    def fill_o(i, _):
        ones_v[pl.ds(i * 16, 16)] = jnp.ones((16,), jnp.float32)
        return 0

    lax.fori_loop(0, EB // 16, fill_o, 0)

    pltpu.sync_copy(zb_v, deg_sp.at[pl.ds(t * 656, 656)])
    plsc.subcore_barrier()

    base = (c * NS + t) * (E_PAD // (NC * NS))

    def body(i, _):
        pltpu.sync_copy(dst_hbm.at[pl.ds(base + i * EB, EB)], idx_v)
        pltpu.sync_copy(ones_v, deg_sp.at[idx_v], add=True)
        return 0

    lax.fori_loop(0, (E_PAD // (NC * NS)) // EB, body, 0)
    plsc.subcore_barrier()
    pltpu.sync_copy(deg_sp.at[pl.ds(t * 656, 656)], zb_v)
    pltpu.sync_copy(zb_v, deg_out.at[pl.ds(c * DEG_SIZE + t * 656, 656)])


_deg_kernel = pl.kernel(
    _deg_body,
    out_type=jax.ShapeDtypeStruct((NC * DEG_SIZE,), jnp.float32),
    mesh=plsc.VectorSubcoreMesh(core_axis_name="c", subcore_axis_name="s"),
    scratch_types=[
        pltpu.VMEM((EB,), jnp.int32),
        pltpu.VMEM((EB,), jnp.float32),
        pltpu.VMEM((656,), jnp.float32),
        pltpu.VMEM_SHARED((DEG_SIZE,), jnp.float32),
    ],
)


NB = (E_PAD // NS) // EB


def _agg_body(s_hbm, srcx_hbm, dst_hbm, acc_out,
              sb0, sb1, sb2, sb3, db0, db1, db2, db3,
              rb0, rb1, rb2, rb3, acc_sp,
              gs0, gs1, gs2, gs3, ss0, ss1, ss2, ss3, is0, is1, is2, is3):
    c = lax.axis_index("c")
    t = lax.axis_index("s")
    sb = [sb0, sb1, sb2, sb3]
    db = [db0, db1, db2, db3]
    rb = [rb0, rb1, rb2, rb3]
    gsem = [gs0, gs1, gs2, gs3]
    ssem = [ss0, ss1, ss2, ss3]
    isem = [is0, is1, is2, is3]
    sbase = (c * NS + t) * (E_PAD // NS)
    dbase = t * (E_PAD // NS)

    def start_idx(i, b):
        pltpu.async_copy(srcx_hbm.at[pl.ds(sbase + i * EB, EB)], sb[b], isem[b])
        pltpu.async_copy(dst_hbm.at[pl.ds(dbase + i * EB, EB)], db[b], isem[b])

    def wait_idx(b):
        pltpu.make_async_copy(srcx_hbm.at[pl.ds(sbase, EB)], sb[b], isem[b]).wait()
        pltpu.make_async_copy(dst_hbm.at[pl.ds(dbase, EB)], db[b], isem[b]).wait()

    def start_g(b):
        pltpu.async_copy(s_hbm.at[sb[b]], rb[b], gsem[b])

    def wait_g(b):
        pltpu.make_async_copy(s_hbm.at[sb[b]], rb[b], gsem[b]).wait()

    def start_s(b):
        pltpu.async_copy(rb[b], acc_sp.at[db[b]], ssem[b], add=True)

    def wait_s(b):
        pltpu.make_async_copy(rb[b], acc_sp.at[db[b]], ssem[b]).wait()

    def fill_z(i, _):
        for j in range(DH // 16):
            rb0[i, pl.ds(j * 16, 16)] = jnp.zeros((16,), jnp.float32)
        return 0

    lax.fori_loop(0, EB, fill_z, 0)
    for k in range(8):
        pltpu.sync_copy(rb0, acc_sp.at[pl.ds(t * 648 + k * EB, EB)])
    pltpu.sync_copy(rb0.at[pl.ds(0, 8)], acc_sp.at[pl.ds(t * 648 + 640, 8)])
    plsc.subcore_barrier()

    start_idx(0, 0)
    start_idx(1, 1)
    start_idx(2, 2)
    wait_idx(0)
    start_g(0)
    wait_idx(1)
    start_g(1)

    def step(i, b):
        wait_g(b)
        start_s(b)

        @pl.when(i + 2 < NB)
        def _():
            wait_idx((b + 2) % 4)
            start_g((b + 2) % 4)

        @pl.when(i + 3 < NB)
        def _():
            @pl.when(i >= 1)
            def _():
                wait_s((b + 3) % 4)
            start_idx(i + 3, (b + 3) % 4)

    def body(k, _):
        for b in range(4):
            step(4 * k + b, b)
        return 0

    lax.fori_loop(0, NB // 4, body, 0)
    for b in range(4):
        wait_s(b)
    plsc.subcore_barrier()
    pltpu.sync_copy(acc_sp.at[pl.ds(t * 640, 640)],
                    acc_out.at[pl.ds(c * N_PAD + t * 640, 640)])


_agg_kernel = pl.kernel(
    _agg_body,
    out_type=jax.ShapeDtypeStruct((NC * N_PAD, DH), jnp.float32),
    mesh=plsc.VectorSubcoreMesh(core_axis_name="c", subcore_axis_name="s"),
    scratch_types=(
        [pltpu.VMEM((EB,), jnp.int32) for _ in range(8)]
        + [pltpu.VMEM((EB, DH), jnp.float32) for _ in range(4)]
        + [pltpu.VMEM_SHARED((ACC_ROWS, DH), jnp.float32)]
        + [pltpu.SemaphoreType.DMA for _ in range(12)]
    ),
)


def _k1_body(x_ref, w_ref, degp_ref, s_ref, dinv_ref):
    deg = degp_ref[0, :] + degp_ref[1, :] + 1.0
    dinv = lax.rsqrt(deg)
    z = jnp.dot(x_ref[...], w_ref[...], preferred_element_type=jnp.float32)
    s_ref[...] = z * dinv[:, None]
    dinv_ref[...] = dinv[None, :]


def _k1(x_p, W1, degp):
    return pl.pallas_call(
        _k1_body,
        grid=(GR, 2),
        in_specs=[
            pl.BlockSpec((RB, D), lambda i, j: (i, 0)),
            pl.BlockSpec((D, DH), lambda i, j: (0, j)),
            pl.BlockSpec((2, RB), lambda i, j: (0, i)),
        ],
        out_specs=[
            pl.BlockSpec((RB, DH), lambda i, j: (j * GR + i, 0)),
            pl.BlockSpec((1, RB), lambda i, j: (0, i)),
        ],
        out_shape=[
            jax.ShapeDtypeStruct((NC * N_PAD, DH), jnp.float32),
            jax.ShapeDtypeStruct((1, N_PAD), jnp.float32),
        ],
    )(x_p, W1, degp)


def _k2_body(accA, accB, sA, sB, dinv_ref, b_ref, w_ref, out_ref):
    di = dinv_ref[0, :]
    hA = _elu(di[:, None] * (accA[...] + sA[...]) + b_ref[0, 0:DH][None, :])
    hB = _elu(di[:, None] * (accB[...] + sB[...]) + b_ref[0, DH:D][None, :])
    h = jnp.concatenate([hA, hB], axis=1)
    z = jnp.dot(h, w_ref[...], preferred_element_type=jnp.float32)
    out_ref[...] = z * di[:, None]


def _k2(acc1, s1, dinv, b1, W2):
    blk = pl.BlockSpec((RB, DH), lambda i, j: (i, 0))
    blkB = pl.BlockSpec((RB, DH), lambda i, j: (GR + i, 0))
    return pl.pallas_call(
        _k2_body,
        grid=(GR, 2),
        in_specs=[
            blk, blkB, blk, blkB,
            pl.BlockSpec((1, RB), lambda i, j: (0, i)),
            pl.BlockSpec((1, D), lambda i, j: (0, 0)),
            pl.BlockSpec((D, DH), lambda i, j: (0, j)),
        ],
        out_specs=pl.BlockSpec((RB, DH), lambda i, j: (j * GR + i, 0)),
        out_shape=jax.ShapeDtypeStruct((NC * N_PAD, DH), jnp.float32),
    )(acc1, acc1, s1, s1, dinv, b1, W2)


def _k3_body(accA, accB, sA, sB, dinv_ref, b_ref, out_ref):
    di = dinv_ref[0, :]
    hA = _elu(di[:, None] * (accA[...] + sA[...]) + b_ref[0, 0:DH][None, :])
    hB = _elu(di[:, None] * (accB[...] + sB[...]) + b_ref[0, DH:D][None, :])
    out_ref[...] = jnp.concatenate([hA, hB], axis=1)


def _k3(acc2, s2, dinv, b2):
    blk = pl.BlockSpec((RB, DH), lambda i: (i, 0))
    blkB = pl.BlockSpec((RB, DH), lambda i: (GR + i, 0))
    return pl.pallas_call(
        _k3_body,
        grid=(GR,),
        in_specs=[
            blk, blkB, blk, blkB,
            pl.BlockSpec((1, RB), lambda i: (0, i)),
            pl.BlockSpec((1, D), lambda i: (0, 0)),
        ],
        out_specs=pl.BlockSpec((RB, D), lambda i: (i, 0)),
        out_shape=jax.ShapeDtypeStruct((N_PAD, D), jnp.float32),
    )(acc2, acc2, s2, s2, dinv, b2)


@jax.jit
def _run(x, src, dst, W1, b1, W2, b2):
    n_e = src.shape[0]
    pe = E_PAD - n_e
    src_p = jnp.concatenate([src, jnp.zeros((pe,), jnp.int32)])
    dst_p = jnp.concatenate([dst, jnp.full((pe,), N_PAD, jnp.int32)])
    srcx = jnp.concatenate([src_p, src_p + N_PAD])
    x_p = jnp.pad(x, ((0, N_PAD - x.shape[0]), (0, 0)))

    degp = _deg_kernel(dst_p).reshape(NC, DEG_SIZE)[:, :N_PAD]
    s1, dinv = _k1(x_p, W1, degp)
    acc1 = _agg_kernel(s1, srcx, dst_p)
    s2 = _k2(acc1, s1, dinv, b1[None, :], W2)
    acc2 = _agg_kernel(s2, srcx, dst_p)
    out = _k3(acc2, s2, dinv, b2[None, :])
    return out[:N_NODES]


def kernel(x, edge_index, W1, b1, W2, b2):
    src = edge_index[0].astype(jnp.int32)
    dst = edge_index[1].astype(jnp.int32)
    return _run(x, src, dst, W1, b1, W2, b2)
